# baseline (speedup 1.0000x reference)
import functools

import jax
import jax.numpy as jnp
from jax import lax
from jax.experimental import pallas as pl
from jax.experimental.pallas import tpu as pltpu
from jax.experimental.pallas import tpu_sc as plsc

_B = 4096
_D = 128
_NC = 1
_NS = 16
_NW = _NC * _NS
_BW = _B // _NW
_LANES = 16
_NCH = _D // _LANES
_SCALE = 1.0 / (_B * _D)


def _partial_sums_sc(embeddings, labels, centers):
    mesh = plsc.VectorSubcoreMesh(core_axis_name="c", subcore_axis_name="s",
                                  num_cores=_NC)

    @functools.partial(
        pl.kernel,
        out_type=jax.ShapeDtypeStruct((_NW, _LANES), jnp.float32),
        mesh=mesh,
        scratch_types=[
            pltpu.VMEM((_BW,), jnp.int32),
            pltpu.VMEM((_BW, _D), jnp.float32),
            pltpu.VMEM((_BW, _D), jnp.float32),
            pltpu.VMEM((_LANES,), jnp.float32),
            pltpu.SemaphoreType.DMA,
            pltpu.SemaphoreType.DMA,
        ],
    )
    def body(emb_hbm, lab_hbm, ctr_hbm, out_hbm,
             idx_v, ctr_v, emb_v, acc_v, sem_g, sem_e):
        cid = lax.axis_index("c")
        sid = lax.axis_index("s")
        wid = sid * _NC + cid
        base = wid * _BW

        emb_cp = pltpu.async_copy(emb_hbm.at[pl.ds(base, _BW)], emb_v, sem_e)
        pltpu.sync_copy(lab_hbm.at[pl.ds(base, _BW)], idx_v)
        gather = pltpu.async_copy(ctr_hbm.at[idx_v], ctr_v, sem_g)
        emb_cp.wait()
        gather.wait()

        zero = jnp.zeros((_LANES,), jnp.float32)

        def row_body(r, accs):
            new = []
            for c in range(_NCH):
                d = (emb_v[r, pl.ds(c * _LANES, _LANES)]
                     - ctr_v[r, pl.ds(c * _LANES, _LANES)])
                new.append(accs[c] + d * d)
            return tuple(new)

        accs = lax.fori_loop(0, _BW, row_body, (zero,) * _NCH)
        acc = ((accs[0] + accs[1]) + (accs[2] + accs[3])) + \
              ((accs[4] + accs[5]) + (accs[6] + accs[7]))
        acc_v[...] = acc
        pltpu.sync_copy(acc_v, out_hbm.at[wid])

    return body(embeddings, labels, centers)


def _fold_tc(partials):
    def body(p_ref, o_ref):
        o_ref[0, 0] = jnp.sum(p_ref[...]) * _SCALE

    return pl.pallas_call(
        body,
        out_shape=jax.ShapeDtypeStruct((1, 1), jnp.float32),
        out_specs=pl.BlockSpec(memory_space=pltpu.SMEM),
    )(partials)


def kernel(embeddings, labels, centers):
    parts = _partial_sums_sc(embeddings, labels.astype(jnp.int32), centers)
    return _fold_tc(parts)[0, 0]


# speedup vs baseline: 1.0238x; 1.0238x over previous
import functools

import jax
import jax.numpy as jnp
from jax import lax
from jax.experimental import pallas as pl
from jax.experimental.pallas import tpu as pltpu
from jax.experimental.pallas import tpu_sc as plsc

_B = 4096
_D = 128
_NC = 2
_NS = 16
_NW = _NC * _NS
_BW = _B // _NW
_BH = _BW // 2
_LANES = 16
_NCH = _D // _LANES
_SCALE = 1.0 / (_B * _D)


def _partial_sums_sc(embeddings, labels, centers):
    mesh = plsc.VectorSubcoreMesh(core_axis_name="c", subcore_axis_name="s",
                                  num_cores=_NC)

    @functools.partial(
        pl.kernel,
        out_type=jax.ShapeDtypeStruct((_NW, _LANES), jnp.float32),
        mesh=mesh,
        scratch_types=[
            pltpu.VMEM((_BW,), jnp.int32),
            pltpu.VMEM((_BH, _D), jnp.float32),
            pltpu.VMEM((_BH, _D), jnp.float32),
            pltpu.VMEM((_BH, _D), jnp.float32),
            pltpu.VMEM((_BH, _D), jnp.float32),
            pltpu.VMEM((_LANES,), jnp.float32),
            pltpu.SemaphoreType.DMA,
            pltpu.SemaphoreType.DMA,
            pltpu.SemaphoreType.DMA,
            pltpu.SemaphoreType.DMA,
        ],
    )
    def body(emb_hbm, lab_hbm, ctr_hbm, out_hbm,
             idx_v, ctr0_v, ctr1_v, emb0_v, emb1_v, acc_v,
             sem_g0, sem_g1, sem_e0, sem_e1):
        cid = lax.axis_index("c")
        sid = lax.axis_index("s")
        wid = sid * _NC + cid
        base = wid * _BW

        emb0 = pltpu.async_copy(emb_hbm.at[pl.ds(base, _BH)], emb0_v, sem_e0)
        emb1 = pltpu.async_copy(emb_hbm.at[pl.ds(base + _BH, _BH)], emb1_v,
                                sem_e1)
        pltpu.sync_copy(lab_hbm.at[pl.ds(base, _BW)], idx_v)
        g0 = pltpu.async_copy(ctr_hbm.at[idx_v.at[pl.ds(0, _BH)]], ctr0_v,
                              sem_g0)
        g1 = pltpu.async_copy(ctr_hbm.at[idx_v.at[pl.ds(_BH, _BH)]], ctr1_v,
                              sem_g1)

        zero = jnp.zeros((_LANES,), jnp.float32)

        def make_row_body(emb_ref, ctr_ref):
            def row_body(r, accs):
                new = []
                for c in range(_NCH):
                    d = (emb_ref[r, pl.ds(c * _LANES, _LANES)]
                         - ctr_ref[r, pl.ds(c * _LANES, _LANES)])
                    new.append(accs[c] + d * d)
                return tuple(new)
            return row_body

        emb0.wait()
        g0.wait()
        accs = lax.fori_loop(0, _BH, make_row_body(emb0_v, ctr0_v),
                             (zero,) * _NCH)
        emb1.wait()
        g1.wait()
        accs = lax.fori_loop(0, _BH, make_row_body(emb1_v, ctr1_v), accs)

        acc = ((accs[0] + accs[1]) + (accs[2] + accs[3])) + \
              ((accs[4] + accs[5]) + (accs[6] + accs[7]))
        acc_v[...] = acc
        pltpu.sync_copy(acc_v, out_hbm.at[wid])

    return body(embeddings, labels, centers)


def _fold_tc(partials):
    def body(p_ref, o_ref):
        o_ref[0, 0] = jnp.sum(p_ref[...]) * _SCALE

    return pl.pallas_call(
        body,
        out_shape=jax.ShapeDtypeStruct((1, 1), jnp.float32),
        out_specs=pl.BlockSpec(memory_space=pltpu.SMEM),
    )(partials)


def kernel(embeddings, labels, centers):
    parts = _partial_sums_sc(embeddings, labels.astype(jnp.int32), centers)
    return _fold_tc(parts)[0, 0]
